# baseline (speedup 1.0000x reference)
import functools

import jax
import jax.numpy as jnp
from jax import lax
from jax.experimental import pallas as pl
from jax.experimental.pallas import tpu as pltpu
from jax.experimental.pallas import tpu_sc as plsc

_B = 16384
_E = 64
_NROWS = 1000001
_NS = 16
_CH = 512
_NFULL = 999936
_TAIL = _NROWS - _NFULL
_NCHUNKS = _NFULL // _CH
_NCH_W = 123
_NCH_LAST = _NCHUNKS - 15 * _NCH_W
_SPAN = _NCH_W * _CH
_NPAIR = (_NCH_W + 1) // 2
_WLPAD = _B + 32
_ROWCAP = 64
_OUTROWS = _B + 8

_mesh = plsc.VectorSubcoreMesh(core_axis_name="c", subcore_axis_name="s")


def _sget(ref, i):
    return ref[pl.ds(i, 16)][0]


def _lane0():
    return lax.iota(jnp.int32, 16) == 0


def _splat(x):
    return jnp.full((16,), x, jnp.int32)


def _do_table(idx_hbm, tabT_hbm, tail_hbm, out3, idxstage, wl, wl2, cnt, start,
              cur, chunkA, chunkB, tail_v, blkA, blkB, blA, blB,
              semA, semB, scA, scB, ws):
    lo = ws * _SPAN
    is_last = ws == 15
    hi = jnp.where(is_last, _NROWS, lo + _SPAN)
    nc_w = jnp.where(is_last, _NCH_LAST, _NCH_W)

    pltpu.async_copy(tabT_hbm.at[:, pl.ds(lo, _CH)], chunkA, semA)
    pltpu.sync_copy(tail_hbm, tail_v)

    z16 = jnp.zeros((16,), jnp.float32)

    def zrow(k, c):
        for g in range(8):
            plsc.store_scatter(
                blkA, [_splat(k), _splat(0), lax.iota(jnp.int32, 16) + g * 16],
                z16)
            plsc.store_scatter(
                blkB, [_splat(k), _splat(0), lax.iota(jnp.int32, 16) + g * 16],
                z16)
        return c
    lax.fori_loop(0, _ROWCAP, zrow, 0)

    def p1_outer(s8, n_w):
        pltpu.sync_copy(idx_hbm.at[pl.ds(s8 * 2048, 2048)], idxstage)

        def p1_inner(t, off):
            v = idxstage[pl.ds(t * 16, 16)]
            m = (v >= lo) & (v < hi)
            bvec = lax.iota(jnp.int32, 16) + (s8 * 2048 + t * 16)
            w = ((v - lo) << 14) | bvec
            mi = m.astype(jnp.int32)
            pc = plsc.cumsum(mi)
            pos = off + pc - mi
            plsc.store_scatter(wl, [pos], w, mask=m)
            return off + jnp.sum(mi)
        return lax.fori_loop(0, 128, p1_inner, n_w)
    n_w = lax.fori_loop(0, 8, p1_outer, 0)

    for t in range(9):
        cnt[pl.ds(t * 16, 16)] = jnp.zeros((16,), jnp.int32)

    def p1b(j, c):
        w = _sget(wl, j)
        bk = w >> 23
        c0 = _sget(cnt, bk)
        plsc.store_scatter(cnt, [_splat(bk)], _splat(c0 + 1), mask=_lane0())
        return c
    lax.fori_loop(0, n_w, p1b, 0)

    def p1c(t, carry):
        cv = cnt[pl.ds(t * 16, 16)]
        incl = plsc.cumsum(cv)
        start[pl.ds(t * 16, 16)] = carry + incl - cv
        cur[pl.ds(t * 16, 16)] = carry + incl - cv
        return carry + jnp.sum(cv)
    lax.fori_loop(0, 9, p1c, 0)

    def p1d(j, c):
        w = _sget(wl, j)
        bk = w >> 23
        p = _sget(cur, bk)
        plsc.store_scatter(wl2, [_splat(p)], _splat(w), mask=_lane0())
        plsc.store_scatter(cur, [_splat(bk)], _splat(p + 1), mask=_lane0())
        return c
    lax.fori_loop(0, n_w, p1d, 0)

    cvecs = [lax.iota(jnp.int32, 16) + g * 16 for g in range(4)]
    max_off = _NFULL - _CH

    def fetch(ci, buf, sem):
        off = jnp.minimum(lo + ci * _CH, max_off)
        return pltpu.async_copy(tabT_hbm.at[:, pl.ds(off, _CH)], buf, sem)

    def wait_chunk(buf, sem):
        pltpu.make_async_copy(tabT_hbm.at[:, pl.ds(0, _CH)], buf, sem).wait()

    def process(ci, chunk_or_tail, blk, bl, sc, use_tail, first):
        pstart = _sget(start, ci)
        pend = _sget(start, ci + 1)
        if use_tail:
            n_i = jnp.where(ci == nc_w, pend - pstart, 0)
        else:
            n_i = jnp.where(ci < nc_w, pend - pstart, 0)
        rounds = jnp.maximum((n_i + (_ROWCAP - 1)) >> 6, 1)

        def round_body(r, c):
            @pl.when(jnp.logical_or(r > 0, jnp.logical_not(first)))
            def _():
                pltpu.make_async_copy(blk, out3.at[bl], sc).wait()
            for t in range(4):
                bl[pl.ds(t * 16, 16)] = jnp.full((16,), _B, jnp.int32)
            n_round = jnp.minimum(n_i - r * _ROWCAP, _ROWCAP)

            def match(k, c2):
                jj = jnp.minimum(pstart + r * _ROWCAP + k, _B + 8)
                w = _sget(wl2, jj)
                rloc = w >> 14
                b = w & (_B - 1)
                if use_tail:
                    rr = jnp.clip(rloc - nc_w * _CH, 0, _TAIL - 1)
                    rrv = _splat(rr)
                else:
                    rr = jnp.clip(rloc - ci * _CH, 0, _CH - 1)
                    rrv = _splat(rr)
                kv = _splat(k)
                for g in range(4):
                    if use_tail:
                        vals = plsc.load_gather(chunk_or_tail, [rrv, cvecs[g]])
                    else:
                        vals = plsc.load_gather(chunk_or_tail, [cvecs[g], rrv])
                    plsc.store_scatter(
                        blk, [kv, _splat(0),
                              lax.iota(jnp.int32, 16) + g * 16], vals)
                plsc.store_scatter(bl, [kv], _splat(b), mask=_lane0())
                return c2
            lax.fori_loop(0, n_round, match, 0)
            pltpu.async_copy(blk, out3.at[bl], sc)
            return c
        lax.fori_loop(0, rounds, round_body, 0)

    def pair(cp, c):
        c0 = 2 * cp
        c1 = c0 + 1
        fetch(c1, chunkB, semB)
        wait_chunk(chunkA, semA)
        process(c0, chunkA, blkA, blA, scA, False, cp == 0)
        fetch(c0 + 2, chunkA, semA)
        wait_chunk(chunkB, semB)
        process(c1, chunkB, blkB, blB, scB, False, cp == 0)
        return c
    lax.fori_loop(0, _NPAIR, pair, 0)

    wait_chunk(chunkA, semA)
    pltpu.make_async_copy(blkB, out3.at[blB], scB).wait()
    process(nc_w, tail_v, blkA, blA, scA, True, False)
    pltpu.make_async_copy(blkA, out3.at[blA], scA).wait()


@functools.partial(
    pl.kernel,
    out_type=[
        jax.ShapeDtypeStruct((_OUTROWS, 1, 128), jnp.float32),
        jax.ShapeDtypeStruct((_OUTROWS, 1, 128), jnp.float32),
    ],
    mesh=_mesh,
    compiler_params=pltpu.CompilerParams(needs_layout_passes=False),
    scratch_types=[
        pltpu.VMEM((2048,), jnp.int32),
        pltpu.VMEM((_WLPAD,), jnp.int32),
        pltpu.VMEM((_WLPAD,), jnp.int32),
        pltpu.VMEM((144,), jnp.int32),
        pltpu.VMEM((144,), jnp.int32),
        pltpu.VMEM((144,), jnp.int32),
        pltpu.VMEM((_E, _CH), jnp.float32),
        pltpu.VMEM((_E, _CH), jnp.float32),
        pltpu.VMEM((_TAIL, _E), jnp.float32),
        pltpu.VMEM((_ROWCAP, 1, 128), jnp.float32),
        pltpu.VMEM((_ROWCAP, 1, 128), jnp.float32),
        pltpu.VMEM((_ROWCAP,), jnp.int32),
        pltpu.VMEM((_ROWCAP,), jnp.int32),
        pltpu.SemaphoreType.DMA,
        pltpu.SemaphoreType.DMA,
        pltpu.SemaphoreType.DMA,
        pltpu.SemaphoreType.DMA,
    ],
)
def _gather_sc(uidx_hbm, iidx_hbm, utT_hbm, itT_hbm, utail_hbm, itail_hbm,
               u3_out, i3_out, idxstage, wl, wl2, cnt, start, cur,
               chunkA, chunkB, tail_v, blkA, blkB, blA, blB,
               semA, semB, scA, scB):
    c = lax.axis_index("c")
    ws = lax.axis_index("s")

    @pl.when(c == 0)
    def _():
        _do_table(uidx_hbm, utT_hbm, utail_hbm, u3_out, idxstage, wl, wl2,
                  cnt, start, cur, chunkA, chunkB, tail_v, blkA, blkB,
                  blA, blB, semA, semB, scA, scB, ws)

    @pl.when(c == 1)
    def _():
        _do_table(iidx_hbm, itT_hbm, itail_hbm, i3_out, idxstage, wl, wl2,
                  cnt, start, cur, chunkA, chunkB, tail_v, blkA, blkB,
                  blA, blB, semA, semB, scA, scB, ws)


def _mlp_body(u_ref, i_ref, w1u_ref, w1i_ref, b1_ref, g1_ref, be1_ref,
              w2_ref, b2_ref, g2_ref, be2_ref, wog_ref, woh_ref, bo_ref,
              out_ref):
    dn = (((1,), (1,)), ((), ()))
    u = u_ref[...][: _B, :]
    it = i_ref[...][: _B, :]
    h = (lax.dot_general(u, w1u_ref[...], dn, preferred_element_type=jnp.float32)
         + lax.dot_general(it, w1i_ref[...], dn, preferred_element_type=jnp.float32)
         + b1_ref[...])
    mu = jnp.mean(h, axis=0, keepdims=True)
    var = jnp.mean((h - mu) ** 2, axis=0, keepdims=True)
    h = (h - mu) * lax.rsqrt(var + 1e-5) * g1_ref[...] + be1_ref[...]
    h = jnp.maximum(h, 0.0)
    h2 = lax.dot_general(h, w2_ref[...], dn, preferred_element_type=jnp.float32) + b2_ref[...]
    mu2 = jnp.mean(h2, axis=0, keepdims=True)
    var2 = jnp.mean((h2 - mu2) ** 2, axis=0, keepdims=True)
    h2 = (h2 - mu2) * lax.rsqrt(var2 + 1e-5) * g2_ref[...] + be2_ref[...]
    h2 = jnp.maximum(h2, 0.0)
    gmf = u * it
    pred = (lax.dot_general(gmf, wog_ref[...], dn, preferred_element_type=jnp.float32)
            + lax.dot_general(h2, woh_ref[...], dn, preferred_element_type=jnp.float32)
            + bo_ref[...])
    out_ref[...] = 4.5 / (1.0 + jnp.exp(-pred)) + 0.5


_mlp = pl.pallas_call(
    _mlp_body,
    out_shape=jax.ShapeDtypeStruct((_B, 1), jnp.float32),
)


def kernel(user_indices, item_indices, user_table, item_table,
           W1, b1, g1, be1, W2, b2, g2, be2, Wo, bo):
    ui = user_indices.astype(jnp.int32)
    ii = item_indices.astype(jnp.int32)
    u3, i3 = _gather_sc(ui, ii, user_table.T, item_table.T,
                        user_table[_NFULL:], item_table[_NFULL:])
    u2 = u3.reshape(_OUTROWS, 128)
    i2 = i3.reshape(_OUTROWS, 128)
    zpad = jnp.zeros((_E, _E), jnp.float32)
    w1u = jnp.concatenate([W1[:, :_E], zpad], axis=1)
    w1i = jnp.concatenate([W1[:, _E:], zpad], axis=1)
    wog = jnp.concatenate([Wo[:, :_E], jnp.zeros((1, _E), jnp.float32)], axis=1)
    pred = _mlp(
        u2, i2, w1u, w1i,
        b1.reshape(1, -1), g1.reshape(1, -1), be1.reshape(1, -1),
        W2,
        b2.reshape(1, -1), g2.reshape(1, -1), be2.reshape(1, -1),
        wog, Wo[:, _E:],
        bo.reshape(1, 1),
    )
    return pred[:, 0]


# speedup vs baseline: 4.2031x; 4.2031x over previous
import functools

import jax
import jax.numpy as jnp
from jax import lax
from jax.experimental import pallas as pl
from jax.experimental.pallas import tpu as pltpu
from jax.experimental.pallas import tpu_sc as plsc

_B = 16384
_E = 64
_BPW = _B // 16
_K = 16

_mesh = plsc.VectorSubcoreMesh(core_axis_name="c", subcore_axis_name="s")


@functools.partial(
    pl.kernel,
    out_type=[
        jax.ShapeDtypeStruct((_B, _E), jnp.float32),
        jax.ShapeDtypeStruct((_B, _E), jnp.float32),
    ],
    mesh=_mesh,
    scratch_types=[
        pltpu.VMEM((_BPW + 16,), jnp.int32),
        pltpu.SemaphoreType.DMA,
    ],
)
def _gather_sc(uidx_hbm, iidx_hbm, utab_hbm, itab_hbm, u_out, i_out,
               idx_v, sem):
    c = lax.axis_index("c")
    ws = lax.axis_index("s")
    base = ws * _BPW

    def do(idx_hbm, tab_hbm, out_hbm):
        pltpu.sync_copy(idx_hbm.at[pl.ds(base, _BPW)],
                        idx_v.at[pl.ds(0, _BPW)])

        def outer(o, carry):
            copies = []
            for t in range(_K):
                j = o * _K + t
                r = idx_v[pl.ds(j, 16)][0]
                copies.append(pltpu.async_copy(
                    tab_hbm.at[pl.ds(r, 1)],
                    out_hbm.at[pl.ds(base + j, 1)], sem))
            for cp in copies:
                cp.wait()
            return carry
        lax.fori_loop(0, _BPW // _K, outer, 0)

    @pl.when(c == 0)
    def _():
        do(uidx_hbm, utab_hbm, u_out)

    @pl.when(c == 1)
    def _():
        do(iidx_hbm, itab_hbm, i_out)


def _mlp_body(u_ref, i_ref, w1u_ref, w1i_ref, b1_ref, g1_ref, be1_ref,
              w2_ref, b2_ref, g2_ref, be2_ref, wog_ref, woh_ref, bo_ref,
              out_ref):
    dn = (((1,), (1,)), ((), ()))
    u = u_ref[...]
    it = i_ref[...]
    h = (lax.dot_general(u, w1u_ref[...], dn, preferred_element_type=jnp.float32)
         + lax.dot_general(it, w1i_ref[...], dn, preferred_element_type=jnp.float32)
         + b1_ref[...])
    mu = jnp.mean(h, axis=0, keepdims=True)
    var = jnp.mean((h - mu) ** 2, axis=0, keepdims=True)
    h = (h - mu) * lax.rsqrt(var + 1e-5) * g1_ref[...] + be1_ref[...]
    h = jnp.maximum(h, 0.0)
    h2 = lax.dot_general(h, w2_ref[...], dn, preferred_element_type=jnp.float32) + b2_ref[...]
    mu2 = jnp.mean(h2, axis=0, keepdims=True)
    var2 = jnp.mean((h2 - mu2) ** 2, axis=0, keepdims=True)
    h2 = (h2 - mu2) * lax.rsqrt(var2 + 1e-5) * g2_ref[...] + be2_ref[...]
    h2 = jnp.maximum(h2, 0.0)
    gmf = u * it
    pred = (lax.dot_general(gmf, wog_ref[...], dn, preferred_element_type=jnp.float32)
            + lax.dot_general(h2, woh_ref[...], dn, preferred_element_type=jnp.float32)
            + bo_ref[...])
    out_ref[...] = 4.5 / (1.0 + jnp.exp(-pred)) + 0.5


_mlp = pl.pallas_call(
    _mlp_body,
    out_shape=jax.ShapeDtypeStruct((_B, 1), jnp.float32),
)


def kernel(user_indices, item_indices, user_table, item_table,
           W1, b1, g1, be1, W2, b2, g2, be2, Wo, bo):
    ui = user_indices.astype(jnp.int32)
    ii = item_indices.astype(jnp.int32)
    u, it = _gather_sc(ui, ii, user_table, item_table)
    pred = _mlp(
        u, it, W1[:, :_E], W1[:, _E:],
        b1.reshape(1, -1), g1.reshape(1, -1), be1.reshape(1, -1),
        W2,
        b2.reshape(1, -1), g2.reshape(1, -1), be2.reshape(1, -1),
        Wo[:, :_E], Wo[:, _E:],
        bo.reshape(1, 1),
    )
    return pred[:, 0]


# speedup vs baseline: 4.2052x; 1.0005x over previous
import functools

import jax
import jax.numpy as jnp
from jax import lax
from jax.experimental import pallas as pl
from jax.experimental.pallas import tpu as pltpu
from jax.experimental.pallas import tpu_sc as plsc

_B = 16384
_E = 64
_BPW = _B // 16
_K = 16

_mesh = plsc.VectorSubcoreMesh(core_axis_name="c", subcore_axis_name="s")


@functools.partial(
    pl.kernel,
    out_type=[
        jax.ShapeDtypeStruct((_B, _E), jnp.float32),
        jax.ShapeDtypeStruct((_B, _E), jnp.float32),
    ],
    mesh=_mesh,
    scratch_types=[
        pltpu.VMEM((_BPW + 16,), jnp.int32),
        pltpu.SemaphoreType.DMA,
    ],
)
def _gather_sc(uidx_hbm, iidx_hbm, utab_hbm, itab_hbm, u_out, i_out,
               idx_v, sem):
    c = lax.axis_index("c")
    ws = lax.axis_index("s")
    base = ws * _BPW

    def do(idx_hbm, tab_hbm, out_hbm):
        pltpu.sync_copy(idx_hbm.at[pl.ds(base, _BPW)],
                        idx_v.at[pl.ds(0, _BPW)])

        def fire(o):
            j0 = o * _K
            v = idx_v[pl.ds(j0, _K)]
            for t in range(_K):
                r = v[t]
                pltpu.async_copy(
                    tab_hbm.at[pl.ds(r, 1)],
                    out_hbm.at[pl.ds(base + j0 + t, 1)], sem)

        def drain():
            pltpu.make_async_copy(
                tab_hbm.at[pl.ds(0, _K)],
                out_hbm.at[pl.ds(base, _K)], sem).wait()

        fire(0)
        fire(1)

        def outer(o, carry):
            fire(o)
            drain()
            return carry
        lax.fori_loop(2, _BPW // _K, outer, 0)
        drain()
        drain()

    @pl.when(c == 0)
    def _():
        do(uidx_hbm, utab_hbm, u_out)

    @pl.when(c == 1)
    def _():
        do(iidx_hbm, itab_hbm, i_out)


def _mlp_body(u_ref, i_ref, w1u_ref, w1i_ref, b1_ref, g1_ref, be1_ref,
              w2_ref, b2_ref, g2_ref, be2_ref, wog_ref, woh_ref, bo_ref,
              out_ref):
    dn = (((1,), (1,)), ((), ()))
    u = u_ref[...]
    it = i_ref[...]
    h = (lax.dot_general(u, w1u_ref[...], dn, preferred_element_type=jnp.float32)
         + lax.dot_general(it, w1i_ref[...], dn, preferred_element_type=jnp.float32)
         + b1_ref[...])
    mu = jnp.mean(h, axis=0, keepdims=True)
    var = jnp.mean((h - mu) ** 2, axis=0, keepdims=True)
    h = (h - mu) * lax.rsqrt(var + 1e-5) * g1_ref[...] + be1_ref[...]
    h = jnp.maximum(h, 0.0)
    h2 = lax.dot_general(h, w2_ref[...], dn, preferred_element_type=jnp.float32) + b2_ref[...]
    mu2 = jnp.mean(h2, axis=0, keepdims=True)
    var2 = jnp.mean((h2 - mu2) ** 2, axis=0, keepdims=True)
    h2 = (h2 - mu2) * lax.rsqrt(var2 + 1e-5) * g2_ref[...] + be2_ref[...]
    h2 = jnp.maximum(h2, 0.0)
    gmf = u * it
    pred = (lax.dot_general(gmf, wog_ref[...], dn, preferred_element_type=jnp.float32)
            + lax.dot_general(h2, woh_ref[...], dn, preferred_element_type=jnp.float32)
            + bo_ref[...])
    out_ref[...] = 4.5 / (1.0 + jnp.exp(-pred)) + 0.5


_mlp = pl.pallas_call(
    _mlp_body,
    out_shape=jax.ShapeDtypeStruct((_B, 1), jnp.float32),
)


def kernel(user_indices, item_indices, user_table, item_table,
           W1, b1, g1, be1, W2, b2, g2, be2, Wo, bo):
    ui = user_indices.astype(jnp.int32)
    ii = item_indices.astype(jnp.int32)
    u, it = _gather_sc(ui, ii, user_table, item_table)
    pred = _mlp(
        u, it, W1[:, :_E], W1[:, _E:],
        b1.reshape(1, -1), g1.reshape(1, -1), be1.reshape(1, -1),
        W2,
        b2.reshape(1, -1), g2.reshape(1, -1), be2.reshape(1, -1),
        Wo[:, :_E], Wo[:, _E:],
        bo.reshape(1, 1),
    )
    return pred[:, 0]
